# Initial kernel scaffold; baseline (speedup 1.0000x reference)
import jax
import jax.numpy as jnp
from jax.experimental import pallas as pl


def kernel(input, embedding_weight):
    raise NotImplementedError("write your pallas kernel here")


# speedup vs baseline: 1.0942x; 1.0942x over previous
import jax
import jax.numpy as jnp
from jax import lax
from jax.experimental import pallas as pl
from jax.experimental.pallas import tpu as pltpu
from jax.experimental.pallas import tpu_sc as plsc

_NC = 2
_NS = 16
_NW = _NC * _NS

_CHUNK = 1024
_SUB = 128


def _emb_body(idx_hbm, table_hbm, out_hbm, idx_v, rows_v, gsem):
    n = idx_hbm.shape[0]
    b_per_w = n // _NW
    nblk = b_per_w // _CHUNK
    wid = lax.axis_index("s") * _NC + lax.axis_index("c")
    base0 = wid * b_per_w

    def body(blk, carry):
        base = base0 + blk * _CHUNK
        pltpu.sync_copy(idx_hbm.at[pl.ds(base, _CHUNK)], idx_v)
        copies = []
        for j in range(_CHUNK // _SUB):
            copies.append(
                pltpu.async_copy(
                    table_hbm.at[idx_v.at[pl.ds(j * _SUB, _SUB)]],
                    rows_v.at[pl.ds(j * _SUB, _SUB)],
                    gsem,
                )
            )
        for c in copies:
            c.wait()
        pltpu.sync_copy(rows_v, out_hbm.at[pl.ds(base, _CHUNK)])
        return carry

    lax.fori_loop(0, nblk, body, 0)


def kernel(input, embedding_weight):
    B, L = input.shape
    V, D = embedding_weight.shape
    n = B * L
    idx_flat = input.reshape(n)
    mesh = plsc.VectorSubcoreMesh(core_axis_name="c", subcore_axis_name="s")
    run = pl.kernel(
        _emb_body,
        mesh=mesh,
        out_type=jax.ShapeDtypeStruct((n, D), jnp.float32),
        scratch_types=[
            pltpu.VMEM((_CHUNK,), jnp.int32),
            pltpu.VMEM((_CHUNK, D), jnp.float32),
            pltpu.SemaphoreType.DMA,
        ],
        compiler_params=pltpu.CompilerParams(use_tc_tiling_on_sc=False),
    )
    out = run(idx_flat, embedding_weight)
    return out.reshape(B, L, D)


# speedup vs baseline: 1.1103x; 1.0147x over previous
import jax
import jax.numpy as jnp
from jax import lax
from jax.experimental import pallas as pl
from jax.experimental.pallas import tpu as pltpu
from jax.experimental.pallas import tpu_sc as plsc

_NC = 2
_NS = 16
_NW = _NC * _NS

_CHUNK = 1280
_SUB = 128
_NSUB = _CHUNK // _SUB


def _emb_body(idx_hbm, table_hbm, out_hbm, idx_v, rows_v, gsem0, gsem1, osem0, osem1):
    n = idx_hbm.shape[0]
    b_per_w = n // _NW
    nblk = b_per_w // _CHUNK
    npair = nblk // 2
    wid = lax.axis_index("s") * _NC + lax.axis_index("c")
    base0 = wid * b_per_w
    gsems = (gsem0, gsem1)
    osems = (osem0, osem1)

    def fill(s, base):
        pltpu.sync_copy(idx_hbm.at[pl.ds(base, _CHUNK)], idx_v.at[s])
        return [
            pltpu.async_copy(
                table_hbm.at[idx_v.at[s, pl.ds(j * _SUB, _SUB)]],
                rows_v.at[s, pl.ds(j * _SUB, _SUB)],
                gsems[s],
            )
            for j in range(_NSUB)
        ]

    def flush(s, base, copies):
        for c in copies:
            c.wait()
        pltpu.async_copy(rows_v.at[s], out_hbm.at[pl.ds(base, _CHUNK)], osems[s])

    def out_wait(s, base):
        pltpu.make_async_copy(rows_v.at[s], out_hbm.at[pl.ds(base, _CHUNK)], osems[s]).wait()

    for s in range(2):
        flush(s, base0 + s * _CHUNK, fill(s, base0 + s * _CHUNK))

    def body(pair, carry):
        b0 = base0 + 2 * pair * _CHUNK
        cs = []
        for s in range(2):
            out_wait(s, b0 + s * _CHUNK)
            cs.append(fill(s, b0 + s * _CHUNK))
        for s in range(2):
            flush(s, b0 + s * _CHUNK, cs[s])
        return carry

    lax.fori_loop(1, npair, body, 0)
    for s in range(2):
        out_wait(s, base0 + s * _CHUNK)


def kernel(input, embedding_weight):
    B, L = input.shape
    V, D = embedding_weight.shape
    n = B * L
    idx_flat = input.reshape(n)
    mesh = plsc.VectorSubcoreMesh(core_axis_name="c", subcore_axis_name="s")
    run = pl.kernel(
        _emb_body,
        mesh=mesh,
        out_type=jax.ShapeDtypeStruct((n, D), jnp.float32),
        scratch_types=[
            pltpu.VMEM((2, _CHUNK), jnp.int32),
            pltpu.VMEM((2, _CHUNK, D), jnp.float32),
            pltpu.SemaphoreType.DMA,
            pltpu.SemaphoreType.DMA,
            pltpu.SemaphoreType.DMA,
            pltpu.SemaphoreType.DMA,
        ],
        compiler_params=pltpu.CompilerParams(use_tc_tiling_on_sc=False),
    )
    out = run(idx_flat, embedding_weight)
    return out.reshape(B, L, D)


# speedup vs baseline: 1.1105x; 1.0001x over previous
import jax
import jax.numpy as jnp
from jax import lax
from jax.experimental import pallas as pl
from jax.experimental.pallas import tpu as pltpu
from jax.experimental.pallas import tpu_sc as plsc

_NC = 2
_NS = 16
_NW = _NC * _NS

_CHUNK = 1280
_SUB = 1280
_NSUB = _CHUNK // _SUB


def _emb_body(idx_hbm, table_hbm, out_hbm, idx_v, rows_v, gsem0, gsem1, osem0, osem1):
    n = idx_hbm.shape[0]
    b_per_w = n // _NW
    nblk = b_per_w // _CHUNK
    npair = nblk // 2
    wid = lax.axis_index("s") * _NC + lax.axis_index("c")
    base0 = wid * b_per_w
    gsems = (gsem0, gsem1)
    osems = (osem0, osem1)

    def fill(s, base):
        pltpu.sync_copy(idx_hbm.at[pl.ds(base, _CHUNK)], idx_v.at[s])
        return [
            pltpu.async_copy(
                table_hbm.at[idx_v.at[s, pl.ds(j * _SUB, _SUB)]],
                rows_v.at[s, pl.ds(j * _SUB, _SUB)],
                gsems[s],
            )
            for j in range(_NSUB)
        ]

    def flush(s, base, copies):
        for c in copies:
            c.wait()
        pltpu.async_copy(rows_v.at[s], out_hbm.at[pl.ds(base, _CHUNK)], osems[s])

    def out_wait(s, base):
        pltpu.make_async_copy(rows_v.at[s], out_hbm.at[pl.ds(base, _CHUNK)], osems[s]).wait()

    for s in range(2):
        flush(s, base0 + s * _CHUNK, fill(s, base0 + s * _CHUNK))

    def body(pair, carry):
        b0 = base0 + 2 * pair * _CHUNK
        cs = []
        for s in range(2):
            out_wait(s, b0 + s * _CHUNK)
            cs.append(fill(s, b0 + s * _CHUNK))
        for s in range(2):
            flush(s, b0 + s * _CHUNK, cs[s])
        return carry

    lax.fori_loop(1, npair, body, 0)
    for s in range(2):
        out_wait(s, base0 + s * _CHUNK)


def kernel(input, embedding_weight):
    B, L = input.shape
    V, D = embedding_weight.shape
    n = B * L
    idx_flat = input.reshape(n)
    mesh = plsc.VectorSubcoreMesh(core_axis_name="c", subcore_axis_name="s")
    run = pl.kernel(
        _emb_body,
        mesh=mesh,
        out_type=jax.ShapeDtypeStruct((n, D), jnp.float32),
        scratch_types=[
            pltpu.VMEM((2, _CHUNK), jnp.int32),
            pltpu.VMEM((2, _CHUNK, D), jnp.float32),
            pltpu.SemaphoreType.DMA,
            pltpu.SemaphoreType.DMA,
            pltpu.SemaphoreType.DMA,
            pltpu.SemaphoreType.DMA,
        ],
        compiler_params=pltpu.CompilerParams(use_tc_tiling_on_sc=False),
    )
    out = run(idx_flat, embedding_weight)
    return out.reshape(B, L, D)
